# Initial kernel scaffold; baseline (speedup 1.0000x reference)
import jax
import jax.numpy as jnp
from jax.experimental import pallas as pl


def kernel(features, edge_index, edge_values, W):
    raise NotImplementedError("write your pallas kernel here")


# speedup vs baseline: 1.1769x; 1.1769x over previous
import functools

import jax
import jax.numpy as jnp
from jax import lax
from jax.experimental import pallas as pl
from jax.experimental.pallas import tpu as pltpu
from jax.experimental.pallas import tpu_sc as plsc

N_NODES = 10000
N_EDGES = 160000
D = 256
DROP_RATE = 0.2

NW = 32
NB = 313
NPAD = NW * NB
S = 1600
NSTAGES = N_EDGES // S
C = 3072
T = C - S
NBLK = C // 16


def _mm_body(x_ref, w_ref, o_ref):
    o_ref[...] = jnp.dot(x_ref[...], w_ref[...],
                         preferred_element_type=jnp.float32)


def _matmul(x, w):
    return pl.pallas_call(
        _mm_body,
        grid=(10,),
        in_specs=[pl.BlockSpec((1000, D), lambda i: (i, 0)),
                  pl.BlockSpec((D, D), lambda i: (0, 0))],
        out_specs=pl.BlockSpec((1000, D), lambda i: (i, 0)),
        out_shape=jax.ShapeDtypeStruct((N_NODES, D), jnp.float32),
    )(x, w)


def _spmm(rows, cols, vals, maskf, h):
    mesh = plsc.VectorSubcoreMesh(core_axis_name="c", subcore_axis_name="s")

    @functools.partial(
        pl.kernel,
        mesh=mesh,
        compiler_params=pltpu.CompilerParams(needs_layout_passes=False),
        out_type=jax.ShapeDtypeStruct((NPAD * D,), jnp.float32),
        scratch_types=[
            pltpu.VMEM((S,), jnp.int32),
            pltpu.VMEM((S,), jnp.int32),
            pltpu.VMEM((S,), jnp.float32),
            pltpu.VMEM((S,), jnp.float32),
            pltpu.VMEM((C + 16,), jnp.int32),
            pltpu.VMEM((C + 16,), jnp.int32),
            pltpu.VMEM((C + 16,), jnp.float32),
            pltpu.VMEM((16, D), jnp.float32),
            pltpu.VMEM((NB * D,), jnp.float32),
            pltpu.SemaphoreType.DMA,
        ],
    )
    def k(rows_hbm, cols_hbm, vals_hbm, mask_hbm, h_hbm, out_hbm,
          rows_s, cols_s, vals_s, mask_s,
          rows_b, cols_b, vals_b, gbuf, acc, sem):
        wid = lax.axis_index("s") * 2 + lax.axis_index("c")
        lo = wid * NB
        hi = lo + NB
        zero16f = jnp.zeros((16,), jnp.float32)
        zero16i = jnp.zeros((16,), jnp.int32)

        def zbody(i, c):
            acc[pl.ds(i * 16, 16)] = zero16f
            return c
        lax.fori_loop(0, NB * D // 16, zbody, 0)

        def do_flush(cnt):
            cols_b[pl.ds(cnt, 16)] = zero16i
            rows_b[pl.ds(cnt, 16)] = zero16i
            vals_b[pl.ds(cnt, 16)] = zero16f

            def blk(j, c):
                @pl.when(j * 16 < cnt)
                def _():
                    off = j * 16
                    idxv = cols_b[pl.ds(off, 16)]
                    pltpu.async_copy(h_hbm.at[idxv], gbuf, sem).wait()
                    vv = vals_b[pl.ds(off, 16)]
                    rr = rows_b[pl.ds(off, 16)]
                    for e in range(16):
                        v = vv[e]
                        base = rr[e] * D
                        for q in range(16):
                            plsc.addupdate(
                                acc.at[pl.ds(base + q * 16, 16)],
                                v * gbuf[e, pl.ds(q * 16, 16)])
                return c
            lax.fori_loop(0, NBLK, blk, 0)

        def filt(v, cnt):
            off = v * 16
            rv = rows_s[pl.ds(off, 16)]
            m = (rv >= lo) & (rv < hi)
            cv = cols_s[pl.ds(off, 16)]
            vmv = vals_s[pl.ds(off, 16)] * mask_s[pl.ds(off, 16)]
            pos = plsc.cumsum(m.astype(jnp.int32))
            idx = cnt + pos - 1
            plsc.store_scatter(cols_b, [idx], cv, mask=m)
            plsc.store_scatter(rows_b, [idx], rv - lo, mask=m)
            plsc.store_scatter(vals_b, [idx], vmv, mask=m)
            return cnt + pos[15]

        def stage(si, cnt):
            soff = si * S
            pltpu.sync_copy(rows_hbm.at[pl.ds(soff, S)], rows_s)
            pltpu.sync_copy(cols_hbm.at[pl.ds(soff, S)], cols_s)
            pltpu.sync_copy(vals_hbm.at[pl.ds(soff, S)], vals_s)
            pltpu.sync_copy(mask_hbm.at[pl.ds(soff, S)], mask_s)
            cnt = lax.fori_loop(0, S // 16, filt, cnt)
            flush = (cnt >= T) | (si == NSTAGES - 1)

            @pl.when(flush)
            def _():
                do_flush(cnt)
            return jnp.where(flush, jnp.int32(0), cnt)

        lax.fori_loop(0, NSTAGES, stage, jnp.int32(0))
        pltpu.sync_copy(acc, out_hbm.at[pl.ds(lo * D, NB * D)])

    return k(rows, cols, vals, maskf, h)


def kernel(features, edge_index, edge_values, W):
    nnz = edge_values.shape[0]
    drop_num = int(nnz * DROP_RATE)
    drop_key = jax.random.key(42)
    drop_idx = jax.random.randint(drop_key, (drop_num,), 0, nnz)
    maskf = jnp.ones((nnz,), jnp.float32).at[drop_idx].set(0.0)

    rows = edge_index[0].astype(jnp.int32)
    cols = edge_index[1].astype(jnp.int32)
    vals = edge_values.astype(jnp.float32)

    h = _matmul(features, W)
    out_flat = _spmm(rows, cols, vals, maskf, h)
    return out_flat.reshape(NPAD, D)[:N_NODES]


# speedup vs baseline: 2.2630x; 1.9229x over previous
import functools

import jax
import jax.numpy as jnp
from jax import lax
from jax.experimental import pallas as pl
from jax.experimental.pallas import tpu as pltpu
from jax.experimental.pallas import tpu_sc as plsc

N_NODES = 10000
N_EDGES = 160000
D = 256
DROP_RATE = 0.2

NW = 32
NB = 313
NPAD = NW * NB
S = 1600
NSTAGES = N_EDGES // S
C = 3072
T = C - S
G = 32
NGRP = C // G


def _mm_body(x_ref, w_ref, o_ref):
    o_ref[...] = jnp.dot(x_ref[...], w_ref[...],
                         preferred_element_type=jnp.float32)


def _matmul(x, w):
    return pl.pallas_call(
        _mm_body,
        grid=(10,),
        in_specs=[pl.BlockSpec((1000, D), lambda i: (i, 0)),
                  pl.BlockSpec((D, D), lambda i: (0, 0))],
        out_specs=pl.BlockSpec((1000, D), lambda i: (i, 0)),
        out_shape=jax.ShapeDtypeStruct((N_NODES, D), jnp.float32),
    )(x, w)


def _vm_body(v_ref, m_ref, o_ref):
    o_ref[...] = v_ref[...] * m_ref[...]


def _edge_mask_mul(vals, maskf):
    v2 = vals.reshape(1250, 128)
    m2 = maskf.reshape(1250, 128)
    out = pl.pallas_call(
        _vm_body,
        out_shape=jax.ShapeDtypeStruct((1250, 128), jnp.float32),
    )(v2, m2)
    return out.reshape(N_EDGES)


def _spmm(rows, cols, vm, h):
    mesh = plsc.VectorSubcoreMesh(core_axis_name="c", subcore_axis_name="s")

    @functools.partial(
        pl.kernel,
        mesh=mesh,
        compiler_params=pltpu.CompilerParams(needs_layout_passes=False),
        out_type=jax.ShapeDtypeStruct((NPAD * D,), jnp.float32),
        scratch_types=[
            pltpu.VMEM((S,), jnp.int32),
            pltpu.VMEM((S,), jnp.int32),
            pltpu.VMEM((S,), jnp.int32),
            pltpu.VMEM((S,), jnp.int32),
            pltpu.VMEM((S,), jnp.float32),
            pltpu.VMEM((S,), jnp.float32),
            pltpu.VMEM((C + 64,), jnp.int32),
            pltpu.VMEM((C + 64,), jnp.int32),
            pltpu.VMEM((C + 64,), jnp.float32),
            pltpu.VMEM((G, D), jnp.float32),
            pltpu.VMEM((G, D), jnp.float32),
            pltpu.VMEM((NB * D,), jnp.float32),
            pltpu.SemaphoreType.DMA,
            pltpu.SemaphoreType.DMA,
            pltpu.SemaphoreType.DMA,
            pltpu.SemaphoreType.DMA,
        ],
    )
    def k(rows_hbm, cols_hbm, vm_hbm, h_hbm, out_hbm,
          rs0, rs1, cs0, cs1, vs0, vs1, rows_b, cols_b, vals_b,
          gb0, gb1, acc, ss0, ss1, gs0, gs1):
        wid = lax.axis_index("s") * 2 + lax.axis_index("c")
        lo = wid * NB
        hi = lo + NB
        zero16f = jnp.zeros((16,), jnp.float32)
        zero16i = jnp.zeros((16,), jnp.int32)
        gbufs = (gb0, gb1)
        gsems = (gs0, gs1)
        ssems = (ss0, ss1)
        rows_ss = (rs0, rs1)
        cols_ss = (cs0, cs1)
        vm_ss = (vs0, vs1)

        @plsc.parallel_loop(0, NB * D, step=16)
        def _(i):
            acc[pl.ds(i, 16)] = zero16f

        def issue_stage(s_idx, slot):
            soff = s_idx * S
            pltpu.async_copy(rows_hbm.at[pl.ds(soff, S)], rows_ss[slot],
                             ssems[slot])
            pltpu.async_copy(cols_hbm.at[pl.ds(soff, S)], cols_ss[slot],
                             ssems[slot])
            pltpu.async_copy(vm_hbm.at[pl.ds(soff, S)], vm_ss[slot],
                             ssems[slot])

        def wait_stage(slot):
            pltpu.make_async_copy(rows_hbm.at[pl.ds(0, S)], rows_ss[slot],
                                  ssems[slot]).wait()
            pltpu.make_async_copy(cols_hbm.at[pl.ds(0, S)], cols_ss[slot],
                                  ssems[slot]).wait()
            pltpu.make_async_copy(vm_hbm.at[pl.ds(0, S)], vm_ss[slot],
                                  ssems[slot]).wait()

        def issue_gather(blk, slot):
            idx = cols_b.at[pl.ds(blk * G, G)]
            pltpu.async_copy(h_hbm.at[idx], gbufs[slot], gsems[slot])

        def wait_gather(slot):
            pltpu.make_async_copy(h_hbm.at[pl.ds(0, G)], gbufs[slot],
                                  gsems[slot]).wait()

        def do_flush(cnt):
            for p in range(G // 16):
                cols_b[pl.ds(cnt + p * 16, 16)] = zero16i
                rows_b[pl.ds(cnt + p * 16, 16)] = zero16i
                vals_b[pl.ds(cnt + p * 16, 16)] = zero16f
            for b in range(2):
                @pl.when(b * G < cnt)
                def _():
                    issue_gather(b, b)

            def process(j, slot):
                off = j * G
                gb = gbufs[slot]

                @plsc.parallel_loop(0, G)
                def _(e):
                    v = vals_b[pl.ds(off + e, 16)][0]
                    base = rows_b[pl.ds(off + e, 16)][0] * D
                    prods = [v * gb[e, pl.ds(q * 16, 16)] for q in range(16)]
                    for q in range(16):
                        plsc.addupdate(acc.at[pl.ds(base + q * 16, 16)],
                                       prods[q])

            def grp(jo, c):
                for b in range(2):
                    j = jo * 2 + b

                    @pl.when(j * G < cnt)
                    def _():
                        wait_gather(b)
                        process(j, b)

                        @pl.when((j + 2) * G < cnt)
                        def _():
                            issue_gather(j + 2, b)
                return c
            lax.fori_loop(0, NGRP // 2, grp, 0)

        def filt_maker(slot):
            def filt(v, cnt):
                off = v * 16
                rv = rows_ss[slot][pl.ds(off, 16)]
                m = (rv >= lo) & (rv < hi)
                pc = plsc.all_reduce_population_count(m)[0]

                @pl.when(pc > 0)
                def _():
                    cv = cols_ss[slot][pl.ds(off, 16)]
                    vmv = vm_ss[slot][pl.ds(off, 16)]
                    pos = plsc.cumsum(m.astype(jnp.int32))
                    idx = cnt + pos - 1
                    plsc.store_scatter(cols_b, [idx], cv, mask=m)
                    plsc.store_scatter(rows_b, [idx], rv - lo, mask=m)
                    plsc.store_scatter(vals_b, [idx], vmv, mask=m)
                return cnt + pc
            return filt

        issue_stage(0, 0)
        issue_stage(1, 1)

        def stage2(so, cnt):
            for slot in range(2):
                si = so * 2 + slot
                wait_stage(slot)
                cnt = lax.fori_loop(0, S // 16, filt_maker(slot), cnt)

                @pl.when(si + 2 < NSTAGES)
                def _():
                    issue_stage(si + 2, slot)
                flush = (cnt >= T) | (si == NSTAGES - 1)

                @pl.when(flush)
                def _():
                    do_flush(cnt)
                cnt = jnp.where(flush, jnp.int32(0), cnt)
            return cnt

        lax.fori_loop(0, NSTAGES // 2, stage2, jnp.int32(0))
        pltpu.sync_copy(acc, out_hbm.at[pl.ds(lo * D, NB * D)])

    return k(rows, cols, vm, h)


def kernel(features, edge_index, edge_values, W):
    nnz = edge_values.shape[0]
    drop_num = int(nnz * DROP_RATE)
    drop_key = jax.random.key(42)
    drop_idx = jax.random.randint(drop_key, (drop_num,), 0, nnz)
    maskf = jnp.ones((nnz,), jnp.float32).at[drop_idx].set(0.0)

    rows = edge_index[0].astype(jnp.int32)
    cols = edge_index[1].astype(jnp.int32)
    vals = edge_values.astype(jnp.float32)

    vm = _edge_mask_mul(vals, maskf)
    h = _matmul(features, W)
    out_flat = _spmm(rows, cols, vm, h)
    return out_flat.reshape(NPAD, D)[:N_NODES]
